# Initial kernel scaffold; baseline (speedup 1.0000x reference)
import jax
import jax.numpy as jnp
from jax.experimental import pallas as pl


def kernel(x, edge_index, batch, W1, att_src1, att_dst1, b1, W2, att_src2, att_dst2, b2, gn_w, gn_b, gn_ms, lin_w, lin_b):
    raise NotImplementedError("write your pallas kernel here")


# speedup vs baseline: 10.0567x; 10.0567x over previous
import functools

import jax
import jax.numpy as jnp
from jax import lax
from jax.experimental import pallas as pl
from jax.experimental.pallas import tpu as pltpu
from jax.experimental.pallas import tpu_sc as plsc

N = 10000
E = 320000
G = 64
NPAD = 10240
EPAD = 327680
EROWS = EPAD // 128
DUMMY_DST = 10008
R = 1000


def _row_blk(i):
    return (i, 0)


def _fixed(i):
    return (0, 0)


def _tc_dense1(x, wa, wb, asa, asb, ada, adb):
    def body(x_ref, wa_ref, wb_ref, asa_ref, asb_ref, ada_ref, adb_ref,
             ha_ref, hb_ref, asrc_ref, adst_ref, exs_ref):
        xb = x_ref[...]
        ha = jnp.dot(xb, wa_ref[...], preferred_element_type=jnp.float32)
        hb = jnp.dot(xb, wb_ref[...], preferred_element_type=jnp.float32)
        ha_ref[...] = ha
        hb_ref[...] = hb
        asrc = ((ha * asa_ref[...]).sum(-1, keepdims=True)
                + (hb * asb_ref[...]).sum(-1, keepdims=True))
        adst = ((ha * ada_ref[...]).sum(-1, keepdims=True)
                + (hb * adb_ref[...]).sum(-1, keepdims=True))
        asrc_ref[...] = asrc
        adst_ref[...] = adst
        s = asrc + adst
        exs_ref[...] = jnp.exp(jnp.where(s > 0, s, 0.2 * s))

    return pl.pallas_call(
        body,
        grid=(N // R,),
        in_specs=[pl.BlockSpec((R, 128), _row_blk)]
        + [pl.BlockSpec((128, 128), _fixed)] * 2
        + [pl.BlockSpec((1, 128), _fixed)] * 4,
        out_specs=[pl.BlockSpec((R, 128), _row_blk)] * 2
        + [pl.BlockSpec((R, 1), _row_blk)] * 3,
        out_shape=[jax.ShapeDtypeStruct((N, 128), jnp.float32)] * 2
        + [jax.ShapeDtypeStruct((N, 1), jnp.float32)] * 3,
    )(x, wa, wb, asa, asb, ada, adb)


def _tc_dense2(p0a, p1a, p0b, p1b, ha, hb, exs, dinv, b1a, b1b,
               w2a, w2b, as2, ad2):
    def body(p0a_ref, p1a_ref, p0b_ref, p1b_ref, ha_ref, hb_ref, exs_ref,
             dinv_ref, b1a_ref, b1b_ref, w2a_ref, w2b_ref, as2_ref, ad2_ref,
             h2_ref, asrc_ref, adst_ref, exs2_ref):
        cs = exs_ref[...] * dinv_ref[...]
        y1a = p0a_ref[...] + p1a_ref[...] + ha_ref[...] * cs + b1a_ref[...]
        y1b = p0b_ref[...] + p1b_ref[...] + hb_ref[...] * cs + b1b_ref[...]
        y1a = jnp.maximum(y1a, 0.0)
        y1b = jnp.maximum(y1b, 0.0)
        h2 = (jnp.dot(y1a, w2a_ref[...], preferred_element_type=jnp.float32)
              + jnp.dot(y1b, w2b_ref[...], preferred_element_type=jnp.float32))
        h2_ref[...] = h2
        asrc = (h2 * as2_ref[...]).sum(-1, keepdims=True)
        adst = (h2 * ad2_ref[...]).sum(-1, keepdims=True)
        asrc_ref[...] = asrc
        adst_ref[...] = adst
        s = asrc + adst
        exs2_ref[...] = jnp.exp(jnp.where(s > 0, s, 0.2 * s))

    return pl.pallas_call(
        body,
        grid=(N // R,),
        in_specs=[pl.BlockSpec((R, 128), _row_blk)] * 6
        + [pl.BlockSpec((R, 1), _row_blk)] * 2
        + [pl.BlockSpec((1, 128), _fixed)] * 2
        + [pl.BlockSpec((128, 128), _fixed)] * 2
        + [pl.BlockSpec((1, 128), _fixed)] * 2,
        out_specs=[pl.BlockSpec((R, 128), _row_blk)]
        + [pl.BlockSpec((R, 1), _row_blk)] * 3,
        out_shape=[jax.ShapeDtypeStruct((N, 128), jnp.float32)]
        + [jax.ShapeDtypeStruct((N, 1), jnp.float32)] * 3,
    )(p0a, p1a, p0b, p1b, ha, hb, exs, dinv, b1a, b1b, w2a, w2b, as2, ad2)


def _tc_stats(p0, p1, h2, exs2, dinv2, b2, batch_col):
    def body(p0_ref, p1_ref, h2_ref, exs_ref, dinv_ref, b2_ref,
             bcol_ref, y2_ref, s1_ref, q_ref, cnt_ref):
        i = pl.program_id(0)
        cs = exs_ref[...] * dinv_ref[...]
        y2 = p0_ref[...] + p1_ref[...] + h2_ref[...] * cs + b2_ref[...]
        y2_ref[...] = y2

        @pl.when(i == 0)
        def _():
            s1_ref[...] = jnp.zeros_like(s1_ref)
            q_ref[...] = jnp.zeros_like(q_ref)
            cnt_ref[...] = jnp.zeros_like(cnt_ref)

        bcol = bcol_ref[...]
        niota = lax.broadcasted_iota(jnp.int32, (R, G), 1)
        m = jnp.where(bcol == niota, 1.0, 0.0)
        dn = (((0,), (0,)), ((), ()))
        s1_ref[...] += lax.dot_general(m, y2, dn,
                                       preferred_element_type=jnp.float32)
        q_ref[...] += lax.dot_general(m, y2 * y2, dn,
                                      preferred_element_type=jnp.float32)
        cnt_ref[...] += lax.dot_general(m, jnp.ones((R, 1), jnp.float32), dn,
                                        preferred_element_type=jnp.float32)

    return pl.pallas_call(
        body,
        grid=(N // R,),
        in_specs=[pl.BlockSpec((R, 128), _row_blk)] * 3
        + [pl.BlockSpec((R, 1), _row_blk)] * 2
        + [pl.BlockSpec((1, 128), _fixed)]
        + [pl.BlockSpec((R, 1), _row_blk)],
        out_specs=[pl.BlockSpec((R, 128), _row_blk),
                   pl.BlockSpec((G, 128), _fixed),
                   pl.BlockSpec((G, 128), _fixed),
                   pl.BlockSpec((G, 1), _fixed)],
        out_shape=[jax.ShapeDtypeStruct((N, 128), jnp.float32),
                   jax.ShapeDtypeStruct((G, 128), jnp.float32),
                   jax.ShapeDtypeStruct((G, 128), jnp.float32),
                   jax.ShapeDtypeStruct((G, 1), jnp.float32)],
    )(p0, p1, h2, exs2, dinv2, b2, batch_col)


def _tc_head(y2, s1, q, cnt, batch_col, gw, gb, gms, lw, lb):
    def body(y2_ref, s1_ref, q_ref, cnt_ref, bcol_ref,
             gw_ref, gb_ref, gms_ref, lw_ref, lb_ref, out_ref, p_acc):
        i = pl.program_id(0)
        cntm = jnp.maximum(cnt_ref[...], 1.0)
        mean = s1_ref[...] / cntm
        ms = gms_ref[...]
        var = q_ref[...] / cntm - (2.0 * ms - ms * ms) * mean * mean
        rstd = lax.rsqrt(var + 1e-5)

        bcol = bcol_ref[...]
        niota = lax.broadcasted_iota(jnp.int32, (R, G), 1)
        m = jnp.where(bcol == niota, 1.0, 0.0)
        meanb = jnp.dot(m, mean, preferred_element_type=jnp.float32)
        rstdb = jnp.dot(m, rstd, preferred_element_type=jnp.float32)
        h = gw_ref[...] * (y2_ref[...] - ms * meanb) * rstdb + gb_ref[...]
        h = jnp.maximum(h, 0.0)

        @pl.when(i == 0)
        def _():
            p_acc[...] = jnp.zeros_like(p_acc)

        dn = (((0,), (0,)), ((), ()))
        p_acc[...] += lax.dot_general(m, h, dn,
                                      preferred_element_type=jnp.float32)

        @pl.when(i == (N // R) - 1)
        def _():
            pooled = p_acc[...] / cntm
            z = jnp.dot(pooled, lw_ref[...],
                        preferred_element_type=jnp.float32) + lb_ref[...]
            out_ref[...] = jax.nn.sigmoid(z)

    return pl.pallas_call(
        body,
        grid=(N // R,),
        in_specs=[pl.BlockSpec((R, 128), _row_blk),
                  pl.BlockSpec((G, 128), _fixed),
                  pl.BlockSpec((G, 128), _fixed),
                  pl.BlockSpec((G, 1), _fixed),
                  pl.BlockSpec((R, 1), _row_blk),
                  pl.BlockSpec((1, 128), _fixed),
                  pl.BlockSpec((1, 128), _fixed),
                  pl.BlockSpec((1, 128), _fixed),
                  pl.BlockSpec((128, 128), _fixed),
                  pl.BlockSpec((1, 128), _fixed)],
        out_specs=pl.BlockSpec((G, 128), _fixed),
        out_shape=jax.ShapeDtypeStruct((G, 128), jnp.float32),
        scratch_shapes=[pltpu.VMEM((G, 128), jnp.float32)],
    )(y2, s1, q, cnt, batch_col, gw, gb, gms, lw, lb)


ACC_ROWS = 10112


def _make_sc_gat(nslabs):
    mesh = plsc.VectorSubcoreMesh(core_axis_name="c", subcore_axis_name="s")

    out_type = ([jax.ShapeDtypeStruct((NPAD,), jnp.float32)]
                + [jax.ShapeDtypeStruct((NPAD, 128), jnp.float32)] * (2 * nslabs))

    scratch = [
        pltpu.VMEM((NPAD,), jnp.float32),
        pltpu.VMEM((NPAD,), jnp.float32),
        pltpu.VMEM((80, 128), jnp.float32),
        pltpu.VMEM((128,), jnp.int32),
        pltpu.VMEM((128,), jnp.int32),
        pltpu.VMEM((128,), jnp.float32),
        pltpu.VMEM((640,), jnp.float32),
        pltpu.VMEM((640,), jnp.float32),
        pltpu.VMEM((128, 128), jnp.float32),
        pltpu.SemaphoreType.DMA,
        pltpu.VMEM_SHARED((ACC_ROWS, 128), jnp.float32),
        pltpu.VMEM_SHARED((NPAD,), jnp.float32),
    ]

    def body(*refs):
        src1d, dst1d, asrcp, adstp, exsp = refs[:5]
        h_slabs = refs[5:5 + nslabs]
        dinv_out = refs[5 + nslabs]
        outs = refs[6 + nslabs:6 + nslabs + 2 * nslabs]
        (asrc_v, adst_v, ex_all, s1b, d1b, exb, den_v, exs_v, rows_v, sem,
         acc_sh, den_sh) = refs[6 + 3 * nslabs:]

        cid = lax.axis_index("c")
        t = lax.axis_index("s")

        zeros16 = jnp.zeros((16,), jnp.float32)

        pltpu.sync_copy(asrcp, asrc_v)
        pltpu.sync_copy(adstp, adst_v)

        def zero_rows_v():
            for r0 in range(128):
                for c0 in range(8):
                    rows_v[r0, pl.ds(c0 * 16, 16)] = zeros16

        def zero_acc_chunk():
            for blk in range(39):
                pltpu.sync_copy(rows_v.at[pl.ds(0, 16), :],
                                acc_sh.at[pl.ds(t * 632 + blk * 16, 16), :])
            pltpu.sync_copy(rows_v.at[pl.ds(0, 8), :],
                            acc_sh.at[pl.ds(t * 632 + 624, 8), :])

        zero_rows_v()
        zero_acc_chunk()
        for c0 in range(40):
            den_v[pl.ds(c0 * 16, 16)] = zeros16
        chunk = pl.ds(t * 640, 640)
        pltpu.sync_copy(den_v, den_sh.at[chunk])
        plsc.subcore_barrier()

        base_mine = t * 160 + cid * 80
        base_other = t * 160 + (1 - cid) * 80

        def p1_mine(j, carry):
            row = base_mine + j
            pltpu.sync_copy(src1d.at[pl.ds(row * 128, 128)], s1b)
            pltpu.sync_copy(dst1d.at[pl.ds(row * 128, 128)], d1b)
            for i in range(8):
                s = pl.ds(i * 16, 16)
                a = (plsc.load_gather(asrc_v, [s1b[s]])
                     + plsc.load_gather(adst_v, [d1b[s]]))
                a = jnp.where(a > 0, a, 0.2 * a)
                ex = jnp.exp(a)
                exb[s] = ex
                ex_all[j, s] = ex
            pltpu.sync_copy(exb, den_sh.at[d1b], add=True)
            return carry

        def p1_other(j, carry):
            row = base_other + j
            pltpu.sync_copy(src1d.at[pl.ds(row * 128, 128)], s1b)
            pltpu.sync_copy(dst1d.at[pl.ds(row * 128, 128)], d1b)
            for i in range(8):
                s = pl.ds(i * 16, 16)
                a = (plsc.load_gather(asrc_v, [s1b[s]])
                     + plsc.load_gather(adst_v, [d1b[s]]))
                a = jnp.where(a > 0, a, 0.2 * a)
                exb[s] = jnp.exp(a)
            pltpu.sync_copy(exb, den_sh.at[d1b], add=True)
            return carry

        lax.fori_loop(0, 80, p1_mine, 0)
        lax.fori_loop(0, 80, p1_other, 0)
        plsc.subcore_barrier()

        pltpu.sync_copy(den_sh.at[chunk], den_v)
        pltpu.sync_copy(exsp.at[chunk], exs_v)
        for c0 in range(40):
            s = pl.ds(c0 * 16, 16)
            den_v[s] = 1.0 / (den_v[s] + exs_v[s] + 1e-16)
        pltpu.sync_copy(den_v, den_sh.at[chunk])

        @pl.when(cid == 0)
        def _():
            pltpu.sync_copy(den_v, dinv_out.at[chunk])

        plsc.subcore_barrier()
        pltpu.sync_copy(den_sh, asrc_v)

        for sl in range(nslabs):
            def p2_block(j, carry):
                row = base_mine + j
                pltpu.sync_copy(src1d.at[pl.ds(row * 128, 128)], s1b)
                pltpu.sync_copy(dst1d.at[pl.ds(row * 128, 128)], d1b)
                pltpu.async_copy(h_slabs[sl].at[s1b], rows_v, sem).wait()
                for i in range(8):
                    s = pl.ds(i * 16, 16)
                    di = plsc.load_gather(asrc_v, [d1b[s]])
                    cvec = ex_all[j, s] * di
                    for l in range(16):
                        c = cvec[l]
                        k = i * 16 + l
                        for jj in range(8):
                            s2 = pl.ds(jj * 16, 16)
                            rows_v[k, s2] = rows_v[k, s2] * c
                pltpu.sync_copy(rows_v, acc_sh.at[d1b], add=True)
                return carry

            lax.fori_loop(0, 80, p2_block, 0)
            plsc.subcore_barrier()

            rs = pl.ds(t * 632, 632)
            o0 = outs[2 * sl]
            o1 = outs[2 * sl + 1]

            @pl.when(cid == 0)
            def _():
                pltpu.sync_copy(acc_sh.at[rs, :], o0.at[rs, :])

            @pl.when(cid == 1)
            def _():
                pltpu.sync_copy(acc_sh.at[rs, :], o1.at[rs, :])

            if sl + 1 < nslabs:
                zero_rows_v()
                zero_acc_chunk()
                plsc.subcore_barrier()

    return pl.kernel(body, out_type=out_type, mesh=mesh,
                     scratch_types=scratch,
                     compiler_params=pltpu.CompilerParams(
                         needs_layout_passes=False))


def _pad_col(v, width):
    return jnp.pad(v.astype(jnp.float32), (0, width - v.shape[0])).reshape(1, width)


def _pad_node(v):
    return jnp.pad(v.reshape(-1), (0, NPAD - N))


def kernel(x, edge_index, batch, W1, att_src1, att_dst1, b1, W2, att_src2,
           att_dst2, b2, gn_w, gn_b, gn_ms, lin_w, lin_b):
    w1p = jnp.pad(W1, ((0, 0), (0, 256 - W1.shape[1])))
    w1a, w1b = w1p[:, :128], w1p[:, 128:]
    as1p = jnp.pad(att_src1, (0, 256 - att_src1.shape[0]))
    ad1p = jnp.pad(att_dst1, (0, 256 - att_dst1.shape[0]))
    as1a, as1b = as1p[:128].reshape(1, 128), as1p[128:].reshape(1, 128)
    ad1a, ad1b = ad1p[:128].reshape(1, 128), ad1p[128:].reshape(1, 128)
    b1p = jnp.pad(b1, (0, 256 - b1.shape[0]))
    b1a, b1b = b1p[:128].reshape(1, 128), b1p[128:].reshape(1, 128)

    w2p = jnp.pad(W2, ((0, 256 - W2.shape[0]), (0, 128 - W2.shape[1])))
    w2a, w2b = w2p[:128, :], w2p[128:, :]
    as2 = _pad_col(att_src2, 128)
    ad2 = _pad_col(att_dst2, 128)
    b2p = _pad_col(b2, 128)
    gwp = _pad_col(gn_w, 128)
    gbp = _pad_col(gn_b, 128)
    gmsp = _pad_col(gn_ms, 128)
    lwp = jnp.pad(lin_w, ((0, 128 - lin_w.shape[0]), (0, 128 - lin_w.shape[1])))
    lbp = _pad_col(lin_b, 128)

    src1d = jnp.pad(edge_index[0], (0, EPAD - E))
    dst1d = jnp.pad(edge_index[1], (0, EPAD - E), constant_values=DUMMY_DST)
    batch_col = batch.reshape(N, 1)

    h1a, h1b, asrc1, adst1, exs1 = _tc_dense1(x, w1a, w1b, as1a, as1b,
                                              ad1a, ad1b)
    sc1 = _make_sc_gat(2)
    dinv1, o0a, o1a, o0b, o1b = sc1(src1d, dst1d, _pad_node(asrc1),
                                    _pad_node(adst1), _pad_node(exs1),
                                    h1a, h1b)

    h2, asrc2, adst2, exs2 = _tc_dense2(o0a, o1a, o0b, o1b, h1a, h1b, exs1,
                                        dinv1.reshape(NPAD, 1), b1a, b1b,
                                        w2a, w2b, as2, ad2)
    sc2 = _make_sc_gat(1)
    dinv2, o0, o1 = sc2(src1d, dst1d, _pad_node(asrc2), _pad_node(adst2),
                        _pad_node(exs2), h2)

    y2, s1, q, cnt = _tc_stats(o0, o1, h2, exs2, dinv2.reshape(NPAD, 1), b2p,
                               batch_col)
    out = _tc_head(y2, s1, q, cnt, batch_col, gwp, gbp, gmsp, lwp, lbp)
    return out[:, :2]


# speedup vs baseline: 13.5466x; 1.3470x over previous
import functools

import jax
import jax.numpy as jnp
from jax import lax
from jax.experimental import pallas as pl
from jax.experimental.pallas import tpu as pltpu
from jax.experimental.pallas import tpu_sc as plsc

N = 10000
E = 320000
G = 64
NPAD = 10240
EPAD = 327680
EROWS = EPAD // 128
DUMMY_DST = 10008
R = 1000


def _row_blk(i):
    return (i, 0)


def _fixed(i):
    return (0, 0)


def _tc_dense1(x, wa, wb, asa, asb, ada, adb):
    def body(x_ref, wa_ref, wb_ref, asa_ref, asb_ref, ada_ref, adb_ref,
             ha_ref, hb_ref, asrc_ref, adst_ref, exs_ref):
        xb = x_ref[...]
        ha = jnp.dot(xb, wa_ref[...], preferred_element_type=jnp.float32)
        hb = jnp.dot(xb, wb_ref[...], preferred_element_type=jnp.float32)
        ha_ref[...] = ha
        hb_ref[...] = hb
        asrc = ((ha * asa_ref[...]).sum(-1, keepdims=True)
                + (hb * asb_ref[...]).sum(-1, keepdims=True))
        adst = ((ha * ada_ref[...]).sum(-1, keepdims=True)
                + (hb * adb_ref[...]).sum(-1, keepdims=True))
        asrc_ref[...] = asrc
        adst_ref[...] = adst
        s = asrc + adst
        exs_ref[...] = jnp.exp(jnp.where(s > 0, s, 0.2 * s))

    return pl.pallas_call(
        body,
        grid=(N // R,),
        in_specs=[pl.BlockSpec((R, 128), _row_blk)]
        + [pl.BlockSpec((128, 128), _fixed)] * 2
        + [pl.BlockSpec((1, 128), _fixed)] * 4,
        out_specs=[pl.BlockSpec((R, 128), _row_blk)] * 2
        + [pl.BlockSpec((R, 1), _row_blk)] * 3,
        out_shape=[jax.ShapeDtypeStruct((N, 128), jnp.float32)] * 2
        + [jax.ShapeDtypeStruct((N, 1), jnp.float32)] * 3,
    )(x, wa, wb, asa, asb, ada, adb)


def _tc_dense2(p0a, p1a, p0b, p1b, ha, hb, exs, den0, den1, b1a, b1b,
               w2a, w2b, as2, ad2):
    def body(p0a_ref, p1a_ref, p0b_ref, p1b_ref, ha_ref, hb_ref, exs_ref,
             den0_ref, den1_ref, b1a_ref, b1b_ref, w2a_ref, w2b_ref, as2_ref,
             ad2_ref, h2_ref, asrc_ref, adst_ref, exs2_ref):
        exs = exs_ref[...]
        dinv = 1.0 / (den0_ref[...] + den1_ref[...] + exs + 1e-16)
        cs = exs * dinv
        y1a = ((p0a_ref[...] + p1a_ref[...]) * dinv + ha_ref[...] * cs
               + b1a_ref[...])
        y1b = ((p0b_ref[...] + p1b_ref[...]) * dinv + hb_ref[...] * cs
               + b1b_ref[...])
        y1a = jnp.maximum(y1a, 0.0)
        y1b = jnp.maximum(y1b, 0.0)
        h2 = (jnp.dot(y1a, w2a_ref[...], preferred_element_type=jnp.float32)
              + jnp.dot(y1b, w2b_ref[...], preferred_element_type=jnp.float32))
        h2_ref[...] = h2
        asrc = (h2 * as2_ref[...]).sum(-1, keepdims=True)
        adst = (h2 * ad2_ref[...]).sum(-1, keepdims=True)
        asrc_ref[...] = asrc
        adst_ref[...] = adst
        s = asrc + adst
        exs2_ref[...] = jnp.exp(jnp.where(s > 0, s, 0.2 * s))

    return pl.pallas_call(
        body,
        grid=(N // R,),
        in_specs=[pl.BlockSpec((R, 128), _row_blk)] * 6
        + [pl.BlockSpec((R, 1), _row_blk)] * 3
        + [pl.BlockSpec((1, 128), _fixed)] * 2
        + [pl.BlockSpec((128, 128), _fixed)] * 2
        + [pl.BlockSpec((1, 128), _fixed)] * 2,
        out_specs=[pl.BlockSpec((R, 128), _row_blk)]
        + [pl.BlockSpec((R, 1), _row_blk)] * 3,
        out_shape=[jax.ShapeDtypeStruct((N, 128), jnp.float32)]
        + [jax.ShapeDtypeStruct((N, 1), jnp.float32)] * 3,
    )(p0a, p1a, p0b, p1b, ha, hb, exs, den0, den1, b1a, b1b, w2a, w2b,
       as2, ad2)


def _tc_stats(p0, p1, h2, exs2, den0, den1, b2, batch_col):
    def body(p0_ref, p1_ref, h2_ref, exs_ref, den0_ref, den1_ref, b2_ref,
             bcol_ref, y2_ref, s1_ref, q_ref, cnt_ref):
        i = pl.program_id(0)
        exs = exs_ref[...]
        dinv = 1.0 / (den0_ref[...] + den1_ref[...] + exs + 1e-16)
        y2 = ((p0_ref[...] + p1_ref[...]) * dinv + h2_ref[...] * exs * dinv
              + b2_ref[...])
        y2_ref[...] = y2

        @pl.when(i == 0)
        def _():
            s1_ref[...] = jnp.zeros_like(s1_ref)
            q_ref[...] = jnp.zeros_like(q_ref)
            cnt_ref[...] = jnp.zeros_like(cnt_ref)

        bcol = bcol_ref[...]
        niota = lax.broadcasted_iota(jnp.int32, (R, G), 1)
        m = jnp.where(bcol == niota, 1.0, 0.0)
        dn = (((0,), (0,)), ((), ()))
        s1_ref[...] += lax.dot_general(m, y2, dn,
                                       preferred_element_type=jnp.float32)
        q_ref[...] += lax.dot_general(m, y2 * y2, dn,
                                      preferred_element_type=jnp.float32)
        cnt_ref[...] += lax.dot_general(m, jnp.ones((R, 1), jnp.float32), dn,
                                        preferred_element_type=jnp.float32)

    return pl.pallas_call(
        body,
        grid=(N // R,),
        in_specs=[pl.BlockSpec((R, 128), _row_blk)] * 3
        + [pl.BlockSpec((R, 1), _row_blk)] * 3
        + [pl.BlockSpec((1, 128), _fixed)]
        + [pl.BlockSpec((R, 1), _row_blk)],
        out_specs=[pl.BlockSpec((R, 128), _row_blk),
                   pl.BlockSpec((G, 128), _fixed),
                   pl.BlockSpec((G, 128), _fixed),
                   pl.BlockSpec((G, 1), _fixed)],
        out_shape=[jax.ShapeDtypeStruct((N, 128), jnp.float32),
                   jax.ShapeDtypeStruct((G, 128), jnp.float32),
                   jax.ShapeDtypeStruct((G, 128), jnp.float32),
                   jax.ShapeDtypeStruct((G, 1), jnp.float32)],
    )(p0, p1, h2, exs2, den0, den1, b2, batch_col)


def _tc_head(y2, s1, q, cnt, batch_col, gw, gb, gms, lw, lb):
    def body(y2_ref, s1_ref, q_ref, cnt_ref, bcol_ref,
             gw_ref, gb_ref, gms_ref, lw_ref, lb_ref, out_ref, p_acc):
        i = pl.program_id(0)
        cntm = jnp.maximum(cnt_ref[...], 1.0)
        mean = s1_ref[...] / cntm
        ms = gms_ref[...]
        var = q_ref[...] / cntm - (2.0 * ms - ms * ms) * mean * mean
        rstd = lax.rsqrt(var + 1e-5)

        bcol = bcol_ref[...]
        niota = lax.broadcasted_iota(jnp.int32, (R, G), 1)
        m = jnp.where(bcol == niota, 1.0, 0.0)
        meanb = jnp.dot(m, mean, preferred_element_type=jnp.float32)
        rstdb = jnp.dot(m, rstd, preferred_element_type=jnp.float32)
        h = gw_ref[...] * (y2_ref[...] - ms * meanb) * rstdb + gb_ref[...]
        h = jnp.maximum(h, 0.0)

        @pl.when(i == 0)
        def _():
            p_acc[...] = jnp.zeros_like(p_acc)

        dn = (((0,), (0,)), ((), ()))
        p_acc[...] += lax.dot_general(m, h, dn,
                                      preferred_element_type=jnp.float32)

        @pl.when(i == (N // R) - 1)
        def _():
            pooled = p_acc[...] / cntm
            z = jnp.dot(pooled, lw_ref[...],
                        preferred_element_type=jnp.float32) + lb_ref[...]
            out_ref[...] = jax.nn.sigmoid(z)

    return pl.pallas_call(
        body,
        grid=(N // R,),
        in_specs=[pl.BlockSpec((R, 128), _row_blk),
                  pl.BlockSpec((G, 128), _fixed),
                  pl.BlockSpec((G, 128), _fixed),
                  pl.BlockSpec((G, 1), _fixed),
                  pl.BlockSpec((R, 1), _row_blk),
                  pl.BlockSpec((1, 128), _fixed),
                  pl.BlockSpec((1, 128), _fixed),
                  pl.BlockSpec((1, 128), _fixed),
                  pl.BlockSpec((128, 128), _fixed),
                  pl.BlockSpec((1, 128), _fixed)],
        out_specs=pl.BlockSpec((G, 128), _fixed),
        out_shape=jax.ShapeDtypeStruct((G, 128), jnp.float32),
        scratch_shapes=[pltpu.VMEM((G, 128), jnp.float32)],
    )(y2, s1, q, cnt, batch_col, gw, gb, gms, lw, lb)


ACC_ROWS = 10112
NBLK = 160


def _make_sc_gat(nslabs):
    mesh = plsc.VectorSubcoreMesh(core_axis_name="c", subcore_axis_name="s")

    out_type = ([jax.ShapeDtypeStruct((NPAD,), jnp.float32)] * 2
                + [jax.ShapeDtypeStruct((NPAD, 128), jnp.float32)] * (2 * nslabs))

    scratch = [
        pltpu.VMEM((NPAD,), jnp.float32),
        pltpu.VMEM((NPAD,), jnp.float32),
        pltpu.VMEM((2, 64), jnp.int32),
        pltpu.VMEM((64,), jnp.int32),
        pltpu.VMEM((64,), jnp.int32),
        pltpu.VMEM((64,), jnp.float32),
        pltpu.VMEM((64,), jnp.float32),
        pltpu.VMEM((64, 128), jnp.float32),
        pltpu.VMEM((64, 128), jnp.float32),
        pltpu.VMEM((640,), jnp.float32),
        pltpu.SemaphoreType.DMA,
        pltpu.SemaphoreType.DMA,
        pltpu.VMEM_SHARED((ACC_ROWS, 128), jnp.float32),
        pltpu.VMEM_SHARED((NPAD,), jnp.float32),
    ]

    def body(*refs):
        src1d, dst1d, asrcp, adstp = refs[:4]
        h_slabs = refs[4:4 + nslabs]
        den_outs = refs[4 + nslabs:6 + nslabs]
        outs = refs[6 + nslabs:6 + nslabs + 2 * nslabs]
        (asrc_v, adst_v, sidx, didx0, didx1, exbuf0, exbuf1,
         rows0, rows1, den_v, sem0, sem1, acc_sh, den_sh) = refs[6 + 3 * nslabs:]

        didx = (didx0, didx1)
        exbuf = (exbuf0, exbuf1)
        rows = (rows0, rows1)
        sems = (sem0, sem1)

        cid = lax.axis_index("c")
        t = lax.axis_index("s")
        wid = t * 2 + cid
        base_e = wid * (NBLK * 64)

        zeros16 = jnp.zeros((16,), jnp.float32)

        pltpu.sync_copy(asrcp, asrc_v)
        pltpu.sync_copy(adstp, adst_v)

        def zero_rows0():
            for r0 in range(64):
                for c0 in range(8):
                    rows0[r0, pl.ds(c0 * 16, 16)] = zeros16

        def zero_acc_chunk():
            for blk in range(39):
                pltpu.sync_copy(rows0.at[pl.ds(0, 16), :],
                                acc_sh.at[pl.ds(t * 632 + blk * 16, 16), :])
            pltpu.sync_copy(rows0.at[pl.ds(0, 8), :],
                            acc_sh.at[pl.ds(t * 632 + 624, 8), :])

        zero_rows0()
        zero_acc_chunk()
        for c0 in range(40):
            den_v[pl.ds(c0 * 16, 16)] = zeros16
        chunk = pl.ds(t * 640, 640)
        pltpu.sync_copy(den_v, den_sh.at[chunk])
        plsc.subcore_barrier()

        def load_and_fire(p, blk, slab):
            e0 = base_e + blk * 64
            pltpu.sync_copy(src1d.at[pl.ds(e0, 64)], sidx.at[p])
            pltpu.sync_copy(dst1d.at[pl.ds(e0, 64)], didx[p])
            return pltpu.async_copy(h_slabs[slab].at[sidx.at[p]], rows[p],
                                    sems[p])

        def edge_pass(slab, fused):
            load_and_fire(0, 0, slab)
            load_and_fire(1, 1, slab)

            def pair(j, carry):
                for p in range(2):
                    blk = 2 * j + p
                    pltpu.make_async_copy(h_slabs[slab].at[sidx.at[p]],
                                          rows[p], sems[p]).wait()
                    for i in range(4):
                        s = pl.ds(i * 16, 16)
                        a = (plsc.load_gather(asrc_v, [sidx[p, s]])
                             + plsc.load_gather(adst_v, [didx[p][s]]))
                        a = jnp.where(a > 0, a, 0.2 * a)
                        cvec = jnp.exp(a)
                        if fused:
                            exbuf[p][s] = cvec
                        for l in range(16):
                            c = cvec[l]
                            k = i * 16 + l
                            for jj in range(8):
                                s2 = pl.ds(jj * 16, 16)
                                rows[p][k, s2] = rows[p][k, s2] * c
                    if fused:
                        pltpu.sync_copy(exbuf[p], den_sh.at[didx[p]],
                                        add=True)
                    pltpu.sync_copy(rows[p], acc_sh.at[didx[p]], add=True)

                    @pl.when(2 * j + p + 2 < NBLK)
                    def _():
                        load_and_fire(p, blk + 2, slab)

                return carry

            lax.fori_loop(0, NBLK // 2, pair, 0)

        edge_pass(0, fused=True)
        plsc.subcore_barrier()

        @pl.when(cid == 0)
        def _():
            pltpu.sync_copy(den_sh.at[chunk], den_outs[0].at[chunk])
            pltpu.sync_copy(acc_sh.at[pl.ds(t * 632, 632), :],
                            outs[0].at[pl.ds(t * 632, 632), :])

        @pl.when(cid == 1)
        def _():
            pltpu.sync_copy(den_sh.at[chunk], den_outs[1].at[chunk])
            pltpu.sync_copy(acc_sh.at[pl.ds(t * 632, 632), :],
                            outs[1].at[pl.ds(t * 632, 632), :])

        for sl in range(1, nslabs):
            zero_rows0()
            zero_acc_chunk()
            plsc.subcore_barrier()
            edge_pass(sl, fused=False)
            plsc.subcore_barrier()

            o0 = outs[2 * sl]
            o1 = outs[2 * sl + 1]

            @pl.when(cid == 0)
            def _():
                pltpu.sync_copy(acc_sh.at[pl.ds(t * 632, 632), :],
                                o0.at[pl.ds(t * 632, 632), :])

            @pl.when(cid == 1)
            def _():
                pltpu.sync_copy(acc_sh.at[pl.ds(t * 632, 632), :],
                                o1.at[pl.ds(t * 632, 632), :])

    return pl.kernel(body, out_type=out_type, mesh=mesh,
                     scratch_types=scratch,
                     compiler_params=pltpu.CompilerParams(
                         needs_layout_passes=False))


def _pad_col(v, width):
    return jnp.pad(v.astype(jnp.float32), (0, width - v.shape[0])).reshape(1, width)


def _pad_node(v):
    return jnp.pad(v.reshape(-1), (0, NPAD - N))


def kernel(x, edge_index, batch, W1, att_src1, att_dst1, b1, W2, att_src2,
           att_dst2, b2, gn_w, gn_b, gn_ms, lin_w, lin_b):
    w1p = jnp.pad(W1, ((0, 0), (0, 256 - W1.shape[1])))
    w1a, w1b = w1p[:, :128], w1p[:, 128:]
    as1p = jnp.pad(att_src1, (0, 256 - att_src1.shape[0]))
    ad1p = jnp.pad(att_dst1, (0, 256 - att_dst1.shape[0]))
    as1a, as1b = as1p[:128].reshape(1, 128), as1p[128:].reshape(1, 128)
    ad1a, ad1b = ad1p[:128].reshape(1, 128), ad1p[128:].reshape(1, 128)
    b1p = jnp.pad(b1, (0, 256 - b1.shape[0]))
    b1a, b1b = b1p[:128].reshape(1, 128), b1p[128:].reshape(1, 128)

    w2p = jnp.pad(W2, ((0, 256 - W2.shape[0]), (0, 128 - W2.shape[1])))
    w2a, w2b = w2p[:128, :], w2p[128:, :]
    as2 = _pad_col(att_src2, 128)
    ad2 = _pad_col(att_dst2, 128)
    b2p = _pad_col(b2, 128)
    gwp = _pad_col(gn_w, 128)
    gbp = _pad_col(gn_b, 128)
    gmsp = _pad_col(gn_ms, 128)
    lwp = jnp.pad(lin_w, ((0, 128 - lin_w.shape[0]), (0, 128 - lin_w.shape[1])))
    lbp = _pad_col(lin_b, 128)

    src1d = jnp.pad(edge_index[0], (0, EPAD - E))
    dst1d = jnp.pad(edge_index[1], (0, EPAD - E), constant_values=DUMMY_DST)
    batch_col = batch.reshape(N, 1)

    h1a, h1b, asrc1, adst1, exs1 = _tc_dense1(x, w1a, w1b, as1a, as1b,
                                              ad1a, ad1b)
    sc1 = _make_sc_gat(2)
    dn0, dn1, o0a, o1a, o0b, o1b = sc1(src1d, dst1d, _pad_node(asrc1),
                                       _pad_node(adst1), h1a, h1b)

    h2, asrc2, adst2, exs2 = _tc_dense2(o0a, o1a, o0b, o1b, h1a, h1b, exs1,
                                        dn0.reshape(NPAD, 1),
                                        dn1.reshape(NPAD, 1), b1a, b1b,
                                        w2a, w2b, as2, ad2)
    sc2 = _make_sc_gat(1)
    dn0_2, dn1_2, o0, o1 = sc2(src1d, dst1d, _pad_node(asrc2),
                               _pad_node(adst2), h2)

    y2, s1, q, cnt = _tc_stats(o0, o1, h2, exs2, dn0_2.reshape(NPAD, 1),
                               dn1_2.reshape(NPAD, 1), b2p, batch_col)
    out = _tc_head(y2, s1, q, cnt, batch_col, gwp, gbp, gmsp, lwp, lbp)
    return out[:, :2]
